# baseline (speedup 1.0000x reference)
import functools

import jax
import jax.numpy as jnp
import numpy as np
from jax import lax
from jax.experimental import pallas as pl
from jax.experimental.pallas import tpu as pltpu
from jax.experimental.pallas import tpu_sc as plsc

B, G, T, E, H = 8, 64, 512, 128, 1024
NW = 32
PP = 7
NSC = NW * PP
PT = B * G - NSC
TE = T * E
HT = T // 2
HTE = HT * E
EB = E // 16


def _mm_body(h_ref, w_ref, o_ref):
    o_ref[...] = jnp.dot(h_ref[...], w_ref[...],
                         preferred_element_type=jnp.float32)


_tc_matmul = pl.pallas_call(
    _mm_body,
    out_shape=jax.ShapeDtypeStruct((B, E), jnp.float32),
)


def _tc_main_body(emb_ref, inter_ref, atop_ref, alpha_ref, c_ref):
    p = pl.program_id(0)
    bp = p // G
    x = emb_ref[0]
    inter = inter_ref[...]
    beta_all = lax.dot_general(x, inter, (((1,), (1,)), ((), ())),
                               preferred_element_type=jnp.float32)
    bsel = lax.broadcasted_iota(jnp.int32, (T, B), 1) == bp
    beta = jnp.sum(jnp.where(bsel, beta_all, 0.0), axis=1,
                   keepdims=True)
    m = jnp.max(beta)
    ex = jnp.exp(beta - m)
    al = ex * (atop_ref[p, 0] / jnp.sum(ex))
    alpha_ref[...] = al
    ct = lax.dot_general(al, x, (((0,), (0,)), ((), ())),
                         preferred_element_type=jnp.float32)

    @pl.when(p % G == 0)
    def _():
        c_ref[...] = jnp.zeros_like(c_ref)

    c_ref[0:1, :] += ct


_tc_main = pl.pallas_call(
    _tc_main_body,
    grid=(PT,),
    in_specs=[
        pl.BlockSpec((1, T, E), lambda p: (p, 0, 0)),
        pl.BlockSpec((B, E), lambda p: (0, 0)),
        pl.BlockSpec(memory_space=pltpu.SMEM),
    ],
    out_specs=[
        pl.BlockSpec((T, 1), lambda p: (p, 0)),
        pl.BlockSpec((8, E), lambda p: (p // G, 0)),
    ],
    out_shape=[
        jax.ShapeDtypeStruct((PT * T, 1), jnp.float32),
        jax.ShapeDtypeStruct((B * 8, E), jnp.float32),
    ],
    compiler_params=pltpu.CompilerParams(
        dimension_semantics=("arbitrary",)),
)


def _sc_body(emb_hbm, inter_hbm, atop_hbm, alpha_hbm, cpart_hbm,
             h0, h1, h2, inter_v, atop_v, beta_v, c_v,
             s0, s1, s2, s_alpha):
    wid = lax.axis_index("s") * 2 + lax.axis_index("c")
    p_base = PT + wid * PP
    pltpu.sync_copy(atop_hbm, atop_v)

    zero16 = jnp.zeros((16,), jnp.float32)
    iota16 = lax.iota(jnp.int32, 16)
    for eb in range(2 * EB):
        c_v[pl.ds(eb * 16, 16)] = zero16
    b_first = p_base // G

    def dma_start(p, half, buf, sem):
        pltpu.async_copy(
            emb_hbm.at[pl.ds(p * TE + half * HTE, HTE)], buf, sem)

    def dma_wait(p, half, buf, sem):
        pltpu.make_async_copy(
            emb_hbm.at[pl.ds(p * TE + half * HTE, HTE)], buf, sem).wait()

    def stage1_half(buf, beta_off, ivs):
        def tb_body(tb, _):
            base0 = tb * (16 * E)
            betav = zero16
            for lane in range(16):
                roff = base0 + lane * E
                m = [buf[pl.ds(roff + eb * 16, 16)] * ivs[eb]
                     for eb in range(EB)]
                r = ((m[0] + m[1]) + (m[2] + m[3])) + \
                    ((m[4] + m[5]) + (m[6] + m[7]))
                betav = jnp.where(iota16 == lane, jnp.sum(r), betav)
            beta_v[pl.ds(beta_off + tb * 16, 16)] = betav
            return 0

        lax.fori_loop(0, HT // 16, tb_body, 0)

    def softmax_scale(p):
        def mx_body(j, mv):
            return jnp.maximum(mv, beta_v[pl.ds(j * 16, 16)])

        mv = lax.fori_loop(0, T // 16, mx_body,
                           jnp.full((16,), -jnp.inf, jnp.float32))
        ms = jnp.full((16,), jnp.max(mv), jnp.float32)

        def ex_body(j, s):
            ev = jnp.exp(beta_v[pl.ds(j * 16, 16)] - ms)
            beta_v[pl.ds(j * 16, 16)] = ev
            return s + ev

        sv = lax.fori_loop(0, T // 16, ex_body, zero16)
        av16 = atop_v[pl.ds((p // 16) * 16, 16)]
        atop_p = jnp.sum(jnp.where(iota16 == (p % 16), av16, 0.0))
        scs = (jnp.full((16,), atop_p, jnp.float32)
               / jnp.full((16,), jnp.sum(sv), jnp.float32))

        def al_body(j, _):
            beta_v[pl.ds(j * 16, 16)] = beta_v[pl.ds(j * 16, 16)] * scs
            return 0

        lax.fori_loop(0, T // 16, al_body, 0)

    def stage2_half(buf, beta_off, accs):
        def tb2_body(tb, accs):
            av16 = beta_v[pl.ds(beta_off + tb * 16, 16)]
            base = tb * (16 * E)
            accs = list(accs)
            for lane in range(16):
                av = jnp.full((16,), av16[lane], jnp.float32)
                toff = base + lane * E
                for eb in range(EB):
                    accs[eb] = accs[eb] + buf[pl.ds(toff + eb * 16, 16)] * av
            return tuple(accs)

        return lax.fori_loop(0, HT // 16, tb2_body, accs)

    def do_pair(i, lo, hi, nxt, s_lo, s_hi, s_nxt):
        p = p_base + i
        dma_start(p, 1, hi, s_hi)
        bp = p // G
        pltpu.sync_copy(inter_hbm.at[bp], inter_v)
        ivs = [inter_v[pl.ds(eb * 16, 16)] for eb in range(EB)]
        dma_wait(p, 0, lo, s_lo)
        stage1_half(lo, 0, ivs)
        dma_wait(p, 1, hi, s_hi)
        stage1_half(hi, HT, ivs)
        softmax_scale(p)
        pltpu.async_copy(beta_v, alpha_hbm.at[p - PT], s_alpha)
        pn = jnp.minimum(p + 1, B * G - 1)
        dma_start(pn, 0, nxt, s_nxt)
        accs = stage2_half(lo, 0, (zero16,) * EB)
        accs = stage2_half(hi, HT, accs)
        slot = jnp.where(bp == b_first, 0, E)
        for eb in range(EB):
            off = slot + eb * 16
            c_v[pl.ds(off, 16)] = c_v[pl.ds(off, 16)] + accs[eb]
        pltpu.make_async_copy(beta_v, alpha_hbm.at[p - PT], s_alpha).wait()

    dma_start(p_base, 0, h0, s0)
    do_pair(0, h0, h1, h2, s0, s1, s2)

    ring = [(h2, h0, h1, s2, s0, s1),
            (h1, h2, h0, s1, s2, s0),
            (h0, h1, h2, s0, s1, s2)]

    def k_body(k, _):
        i1 = 1 + 3 * k
        for j in range(3):
            do_pair(i1 + j, *ring[j])
        return 0

    lax.fori_loop(0, (PP - 1) // 3, k_body, 0)
    pltpu.make_async_copy(emb_hbm.at[pl.ds(0, HTE)], h2, s2).wait()
    pltpu.sync_copy(c_v, cpart_hbm.at[wid])


_sc_call = functools.partial(
    pl.kernel,
    mesh=plsc.VectorSubcoreMesh(core_axis_name="c", subcore_axis_name="s"),
    compiler_params=pltpu.CompilerParams(needs_layout_passes=False),
    out_type=(
        jax.ShapeDtypeStruct((NSC, T), jnp.float32),
        jax.ShapeDtypeStruct((NW, 2 * E), jnp.float32),
    ),
    scratch_types=[
        pltpu.VMEM((HTE,), jnp.float32),
        pltpu.VMEM((HTE,), jnp.float32),
        pltpu.VMEM((HTE,), jnp.float32),
        pltpu.VMEM((E,), jnp.float32),
        pltpu.VMEM((B * G,), jnp.float32),
        pltpu.VMEM((T,), jnp.float32),
        pltpu.VMEM((2 * E,), jnp.float32),
        pltpu.SemaphoreType.DMA,
        pltpu.SemaphoreType.DMA,
        pltpu.SemaphoreType.DMA,
        pltpu.SemaphoreType.DMA,
    ],
)(_sc_body)

_SLOT_B = np.zeros((NW, 2), dtype=np.int32)
for _w in range(NW):
    _SLOT_B[_w, 0] = (PT + _w * PP) // G
    _SLOT_B[_w, 1] = (PT + (_w + 1) * PP - 1) // G


def kernel(decoder_hidden_state, alpha_graph_attention_top, all_embeddings, W):
    inter = _tc_matmul(decoder_hidden_state, W)
    emb3 = all_embeddings.reshape(B * G, T, E)
    atop_flat = alpha_graph_attention_top.reshape(-1)
    alpha_tc_col, c_tc = _tc_main(emb3, inter, atop_flat.reshape(B * G, 1))
    alpha_sc, c_part = _sc_call(all_embeddings.reshape(-1), inter, atop_flat)
    slots = c_part.reshape(NW * 2, E)
    c = c_tc.reshape(B, 8, E)[:, 0, :].at[
        jnp.asarray(_SLOT_B.reshape(-1))].add(slots)
    alpha = jnp.concatenate(
        [alpha_tc_col.reshape(PT, T), alpha_sc], axis=0).reshape(B, G, T)
    return (c, alpha)


# speedup vs baseline: 1.0749x; 1.0749x over previous
import functools

import jax
import jax.numpy as jnp
import numpy as np
from jax import lax
from jax.experimental import pallas as pl
from jax.experimental.pallas import tpu as pltpu
from jax.experimental.pallas import tpu_sc as plsc

B, G, T, E, H = 8, 64, 512, 128, 1024
NW = 32
PP = 7
NSC = NW * PP
PT = B * G - NSC
TE = T * E
HT = T // 2
HTE = HT * E
EB = E // 16


def _mm_body(h_ref, w_ref, o_ref):
    o_ref[...] = jnp.dot(h_ref[...], w_ref[...],
                         preferred_element_type=jnp.float32)


_tc_matmul = pl.pallas_call(
    _mm_body,
    out_shape=jax.ShapeDtypeStruct((B, E), jnp.float32),
)


NBTC = (PT + G - 1) // G


def _tc_main_body(emb_ref, inter_ref, atop_ref, alpha_ref, c_ref):
    p = pl.program_id(0)
    bp = p // G
    x = emb_ref[0]
    inter = inter_ref[...]
    brow = lax.broadcasted_iota(jnp.int32, (B, E), 0) == bp
    iv_row = jnp.sum(jnp.where(brow, inter, 0.0), axis=0,
                     keepdims=True)
    beta = lax.dot_general(x, iv_row, (((1,), (1,)), ((), ())),
                           preferred_element_type=jnp.float32)
    m = jnp.max(beta)
    ex = jnp.exp(beta - m)
    al = ex * (atop_ref[p, 0] / jnp.sum(ex))
    alpha_ref[...] = al
    ct = lax.dot_general(al, x, (((0,), (0,)), ((), ())),
                         preferred_element_type=jnp.float32)

    @pl.when(p % G == 0)
    def _():
        c_ref[...] = jnp.zeros_like(c_ref)

    c_ref[0:1, :] += ct


_tc_main = pl.pallas_call(
    _tc_main_body,
    grid=(PT,),
    in_specs=[
        pl.BlockSpec((1, T, E), lambda p: (p, 0, 0)),
        pl.BlockSpec((B, E), lambda p: (0, 0)),
        pl.BlockSpec(memory_space=pltpu.SMEM),
    ],
    out_specs=[
        pl.BlockSpec((T, 1), lambda p: (p, 0)),
        pl.BlockSpec((8, E), lambda p: (p // G, 0)),
    ],
    out_shape=[
        jax.ShapeDtypeStruct((PT * T, 1), jnp.float32),
        jax.ShapeDtypeStruct((NBTC * 8, E), jnp.float32),
    ],
    compiler_params=pltpu.CompilerParams(
        dimension_semantics=("arbitrary",),
        fuse_transposed_lhs_in_matmul=True),
)


def _sc_body(emb_hbm, inter_hbm, atop_hbm, alpha_hbm, cpart_hbm,
             h0, h1, h2, inter_v, atop_v, beta_v, c_v,
             s0, s1, s2, s_alpha):
    wid = lax.axis_index("s") * 2 + lax.axis_index("c")
    p_base = PT + wid * PP
    pltpu.sync_copy(atop_hbm, atop_v)

    zero16 = jnp.zeros((16,), jnp.float32)
    iota16 = lax.iota(jnp.int32, 16)
    for eb in range(2 * EB):
        c_v[pl.ds(eb * 16, 16)] = zero16
    b_first = p_base // G

    def dma_start(p, half, buf, sem):
        pltpu.async_copy(
            emb_hbm.at[pl.ds(p * TE + half * HTE, HTE)], buf, sem)

    def dma_wait(p, half, buf, sem):
        pltpu.make_async_copy(
            emb_hbm.at[pl.ds(p * TE + half * HTE, HTE)], buf, sem).wait()

    def stage1_half(buf, beta_off, ivs):
        def tb_body(tb, _):
            base0 = tb * (16 * E)
            betav = zero16
            for lane in range(16):
                roff = base0 + lane * E
                m = [buf[pl.ds(roff + eb * 16, 16)] * ivs[eb]
                     for eb in range(EB)]
                r = ((m[0] + m[1]) + (m[2] + m[3])) + \
                    ((m[4] + m[5]) + (m[6] + m[7]))
                betav = jnp.where(iota16 == lane, jnp.sum(r), betav)
            beta_v[pl.ds(beta_off + tb * 16, 16)] = betav
            return 0

        lax.fori_loop(0, HT // 16, tb_body, 0)

    def softmax_scale(p):
        def mx_body(j, mv):
            return jnp.maximum(mv, beta_v[pl.ds(j * 16, 16)])

        mv = lax.fori_loop(0, T // 16, mx_body,
                           jnp.full((16,), -jnp.inf, jnp.float32))
        ms = jnp.full((16,), jnp.max(mv), jnp.float32)

        def ex_body(j, s):
            ev = jnp.exp(beta_v[pl.ds(j * 16, 16)] - ms)
            beta_v[pl.ds(j * 16, 16)] = ev
            return s + ev

        sv = lax.fori_loop(0, T // 16, ex_body, zero16)
        av16 = atop_v[pl.ds((p // 16) * 16, 16)]
        atop_p = jnp.sum(jnp.where(iota16 == (p % 16), av16, 0.0))
        scs = (jnp.full((16,), atop_p, jnp.float32)
               / jnp.full((16,), jnp.sum(sv), jnp.float32))

        def al_body(j, _):
            beta_v[pl.ds(j * 16, 16)] = beta_v[pl.ds(j * 16, 16)] * scs
            return 0

        lax.fori_loop(0, T // 16, al_body, 0)

    def stage2_half(buf, beta_off, accs):
        def tb2_body(tb, accs):
            av16 = beta_v[pl.ds(beta_off + tb * 16, 16)]
            base = tb * (16 * E)
            accs = list(accs)
            for lane in range(16):
                av = jnp.full((16,), av16[lane], jnp.float32)
                toff = base + lane * E
                for eb in range(EB):
                    accs[eb] = accs[eb] + buf[pl.ds(toff + eb * 16, 16)] * av
            return tuple(accs)

        return lax.fori_loop(0, HT // 16, tb2_body, accs)

    def do_pair(i, lo, hi, nxt, s_lo, s_hi, s_nxt):
        p = p_base + i
        dma_start(p, 1, hi, s_hi)
        bp = p // G
        pltpu.sync_copy(inter_hbm.at[bp], inter_v)
        ivs = [inter_v[pl.ds(eb * 16, 16)] for eb in range(EB)]
        dma_wait(p, 0, lo, s_lo)
        stage1_half(lo, 0, ivs)
        dma_wait(p, 1, hi, s_hi)
        stage1_half(hi, HT, ivs)
        softmax_scale(p)
        pltpu.async_copy(beta_v, alpha_hbm.at[p - PT], s_alpha)
        pn = jnp.minimum(p + 1, B * G - 1)
        dma_start(pn, 0, nxt, s_nxt)
        accs = stage2_half(lo, 0, (zero16,) * EB)
        accs = stage2_half(hi, HT, accs)
        slot = jnp.where(bp == b_first, 0, E)
        for eb in range(EB):
            off = slot + eb * 16
            c_v[pl.ds(off, 16)] = c_v[pl.ds(off, 16)] + accs[eb]
        pltpu.make_async_copy(beta_v, alpha_hbm.at[p - PT], s_alpha).wait()

    dma_start(p_base, 0, h0, s0)
    do_pair(0, h0, h1, h2, s0, s1, s2)

    ring = [(h2, h0, h1, s2, s0, s1),
            (h1, h2, h0, s1, s2, s0),
            (h0, h1, h2, s0, s1, s2)]

    def k_body(k, _):
        i1 = 1 + 3 * k
        for j in range(3):
            do_pair(i1 + j, *ring[j])
        return 0

    lax.fori_loop(0, (PP - 1) // 3, k_body, 0)
    pltpu.make_async_copy(emb_hbm.at[pl.ds(0, HTE)], h2, s2).wait()
    pltpu.sync_copy(c_v, cpart_hbm.at[wid])


_sc_call = functools.partial(
    pl.kernel,
    mesh=plsc.VectorSubcoreMesh(core_axis_name="c", subcore_axis_name="s"),
    compiler_params=pltpu.CompilerParams(needs_layout_passes=False),
    out_type=(
        jax.ShapeDtypeStruct((NSC, T), jnp.float32),
        jax.ShapeDtypeStruct((NW, 2 * E), jnp.float32),
    ),
    scratch_types=[
        pltpu.VMEM((HTE,), jnp.float32),
        pltpu.VMEM((HTE,), jnp.float32),
        pltpu.VMEM((HTE,), jnp.float32),
        pltpu.VMEM((E,), jnp.float32),
        pltpu.VMEM((B * G,), jnp.float32),
        pltpu.VMEM((T,), jnp.float32),
        pltpu.VMEM((2 * E,), jnp.float32),
        pltpu.SemaphoreType.DMA,
        pltpu.SemaphoreType.DMA,
        pltpu.SemaphoreType.DMA,
        pltpu.SemaphoreType.DMA,
    ],
)(_sc_body)

_SLOT_M = np.zeros((B, NW * 2), dtype=np.float32)
for _w in range(NW):
    _SLOT_M[(PT + _w * PP) // G, 2 * _w] = 1.0
    _SLOT_M[(PT + (_w + 1) * PP - 1) // G, 2 * _w + 1] = 1.0


def kernel(decoder_hidden_state, alpha_graph_attention_top, all_embeddings, W):
    inter = _tc_matmul(decoder_hidden_state, W)
    emb3 = all_embeddings.reshape(B * G, T, E)
    atop_flat = alpha_graph_attention_top.reshape(-1)
    alpha_sc, c_part = _sc_call(all_embeddings.reshape(-1), inter, atop_flat)
    alpha_tc_col, c_tc = _tc_main(emb3, inter, atop_flat.reshape(B * G, 1))
    c_base = jnp.concatenate(
        [c_tc.reshape(NBTC, 8, E)[:, 0, :],
         jnp.zeros((B - NBTC, E), jnp.float32)], axis=0)
    c = c_base + jnp.asarray(_SLOT_M) @ c_part.reshape(NW * 2, E)
    alpha = jnp.concatenate(
        [alpha_tc_col.reshape(PT, T), alpha_sc], axis=0).reshape(B, G, T)
    return (c, alpha)


# speedup vs baseline: 2.1697x; 2.0185x over previous
import functools

import jax
import jax.numpy as jnp
from jax import lax
from jax.experimental import pallas as pl
from jax.experimental.pallas import tpu as pltpu
from jax.experimental.pallas import tpu_sc as plsc

B, G, T, E, H = 8, 64, 512, 128, 1024
NW = 32
PP = (B * G) // NW
TE = T * E
HT = T // 2
HTE = HT * E
EB = E // 16


def _mm_body(h_ref, w_ref, o_ref):
    o_ref[...] = jnp.dot(h_ref[...], w_ref[...],
                         preferred_element_type=jnp.float32)


_tc_matmul = pl.pallas_call(
    _mm_body,
    out_shape=jax.ShapeDtypeStruct((B, E), jnp.float32),
)


def _sc_body(emb_hbm, inter_hbm, atop_hbm, alpha_hbm, cpart_hbm,
             h0, h1, h2, inter_v, atop_v, beta_v, c_v,
             s0, s1, s2, s_alpha):
    wid = lax.axis_index("s") * 2 + lax.axis_index("c")
    b = wid // (NW // B)
    pltpu.sync_copy(inter_hbm.at[b], inter_v)
    pltpu.sync_copy(atop_hbm.at[pl.ds(wid * PP, PP)], atop_v)

    zero16 = jnp.zeros((16,), jnp.float32)
    iota16 = lax.iota(jnp.int32, 16)
    for eb in range(EB):
        c_v[pl.ds(eb * 16, 16)] = zero16
    ivs = [inter_v[pl.ds(eb * 16, 16)] for eb in range(EB)]
    atop_reg = atop_v[...]

    def dma_start(p, half, buf, sem):
        pltpu.async_copy(
            emb_hbm.at[pl.ds(p * TE + half * HTE, HTE)], buf, sem)

    def dma_wait(p, half, buf, sem):
        pltpu.make_async_copy(
            emb_hbm.at[pl.ds(p * TE + half * HTE, HTE)], buf, sem).wait()

    def stage1_half(buf, beta_off):
        @plsc.parallel_loop(0, HT // 16, unroll=2)
        def tb_body(tb):
            base0 = tb * (16 * E)
            betav = zero16
            for lane in range(16):
                roff = base0 + lane * E
                m = [buf[pl.ds(roff + eb * 16, 16)] * ivs[eb]
                     for eb in range(EB)]
                r = ((m[0] + m[1]) + (m[2] + m[3])) + \
                    ((m[4] + m[5]) + (m[6] + m[7]))
                betav = jnp.where(iota16 == lane, jnp.sum(r), betav)
            beta_v[pl.ds(beta_off + tb * 16, 16)] = betav

    def softmax_scale(i):
        @plsc.parallel_loop(0, T // 16, unroll=2,
                            carry=jnp.full((16,), -jnp.inf, jnp.float32))
        def mv(j, m):
            return jnp.maximum(m, beta_v[pl.ds(j * 16, 16)])

        ms = jnp.full((16,), jnp.max(mv), jnp.float32)

        @plsc.parallel_loop(0, T // 16, unroll=2, carry=zero16)
        def sv(j, s):
            ev = jnp.exp(beta_v[pl.ds(j * 16, 16)] - ms)
            beta_v[pl.ds(j * 16, 16)] = ev
            return s + ev

        atop_i = jnp.sum(jnp.where(iota16 == i, atop_reg, 0.0))
        scs = (jnp.full((16,), atop_i, jnp.float32)
               / jnp.full((16,), jnp.sum(sv), jnp.float32))

        @plsc.parallel_loop(0, T // 16, unroll=2)
        def _scale(j):
            beta_v[pl.ds(j * 16, 16)] = beta_v[pl.ds(j * 16, 16)] * scs

    def stage2_half(buf, beta_off, accs):
        @plsc.parallel_loop(0, HT // 16, unroll=2, carry=tuple(accs))
        def out_accs(tb, accs):
            av16 = beta_v[pl.ds(beta_off + tb * 16, 16)]
            base = tb * (16 * E)
            accs = list(accs)
            for lane in range(16):
                av = jnp.full((16,), av16[lane], jnp.float32)
                toff = base + lane * E
                for eb in range(EB):
                    accs[eb] = accs[eb] + buf[pl.ds(toff + eb * 16, 16)] * av
            return tuple(accs)

        return out_accs

    def do_pair(i, lo, hi, nxt, s_lo, s_hi, s_nxt):
        p = wid * PP + i
        dma_start(p, 1, hi, s_hi)
        dma_wait(p, 0, lo, s_lo)
        stage1_half(lo, 0)
        dma_wait(p, 1, hi, s_hi)
        stage1_half(hi, HT)
        softmax_scale(i)
        pltpu.async_copy(beta_v, alpha_hbm.at[p], s_alpha)
        pn = jnp.minimum(p + 1, B * G - 1)
        dma_start(pn, 0, nxt, s_nxt)
        accs = stage2_half(lo, 0, (zero16,) * EB)
        accs = stage2_half(hi, HT, accs)
        for eb in range(EB):
            c_v[pl.ds(eb * 16, 16)] = c_v[pl.ds(eb * 16, 16)] + accs[eb]
        pltpu.make_async_copy(beta_v, alpha_hbm.at[p], s_alpha).wait()

    dma_start(wid * PP, 0, h0, s0)
    do_pair(0, h0, h1, h2, s0, s1, s2)

    def k_body(k, _):
        i1 = 1 + 3 * k
        do_pair(i1, h2, h0, h1, s2, s0, s1)
        do_pair(i1 + 1, h1, h2, h0, s1, s2, s0)
        do_pair(i1 + 2, h0, h1, h2, s0, s1, s2)
        return 0

    lax.fori_loop(0, (PP - 1) // 3, k_body, 0)
    pltpu.make_async_copy(
        emb_hbm.at[pl.ds(0, HTE)], h2, s2).wait()
    pltpu.sync_copy(c_v, cpart_hbm.at[wid])


_sc_call = functools.partial(
    pl.kernel,
    mesh=plsc.VectorSubcoreMesh(core_axis_name="c", subcore_axis_name="s"),
    compiler_params=pltpu.CompilerParams(needs_layout_passes=False),
    out_type=(
        jax.ShapeDtypeStruct((B * G, T), jnp.float32),
        jax.ShapeDtypeStruct((NW, E), jnp.float32),
    ),
    scratch_types=[
        pltpu.VMEM((HTE,), jnp.float32),
        pltpu.VMEM((HTE,), jnp.float32),
        pltpu.VMEM((HTE,), jnp.float32),
        pltpu.VMEM((E,), jnp.float32),
        pltpu.VMEM((PP,), jnp.float32),
        pltpu.VMEM((T,), jnp.float32),
        pltpu.VMEM((E,), jnp.float32),
        pltpu.SemaphoreType.DMA,
        pltpu.SemaphoreType.DMA,
        pltpu.SemaphoreType.DMA,
        pltpu.SemaphoreType.DMA,
    ],
)(_sc_body)


def kernel(decoder_hidden_state, alpha_graph_attention_top, all_embeddings, W):
    inter = _tc_matmul(decoder_hidden_state, W)
    emb_flat = all_embeddings.reshape(-1)
    atop_flat = alpha_graph_attention_top.reshape(-1)
    alpha_flat, c_part = _sc_call(emb_flat, inter, atop_flat)
    c = c_part.reshape(B, NW // B, E).sum(axis=1)
    alpha = alpha_flat.reshape(B, G, T)
    return (c, alpha)
